# baseline (speedup 1.0000x reference)
import jax
import jax.numpy as jnp
from jax import lax
from jax.experimental import pallas as pl


def _add_rows_body(x_ref, o_ref):
    row = lax.broadcasted_iota(jnp.int32, (8, 1024), 0)
    bump = jnp.where((row == 0) | (row == 4), jnp.float32(1.0), jnp.float32(0.0))
    o_ref[...] = x_ref[...] + bump


def kernel(x):
    n, s, d = x.shape
    y = x.reshape(n * s, d)
    out = pl.pallas_call(
        _add_rows_body,
        out_shape=jax.ShapeDtypeStruct((n * s, d), x.dtype),
        grid=(1,),
        in_specs=[pl.BlockSpec((8, d), lambda i: (0, 0))],
        out_specs=pl.BlockSpec((8, d), lambda i: (0, 0)),
        input_output_aliases={0: 0},
    )(y)
    return out.reshape(n, s, d)


# speedup vs baseline: 2.2917x; 2.2917x over previous
import jax
import jax.numpy as jnp
from jax import lax
from jax.experimental import pallas as pl


def _scatter_add_body(x_ref, o_ref):
    i0 = lax.broadcasted_iota(jnp.int32, (2, 3, 1024), 0)
    i1 = lax.broadcasted_iota(jnp.int32, (2, 3, 1024), 1)
    hit = ((i0 == 0) & (i1 == 0)) | ((i0 == 1) & (i1 == 1))
    o_ref[...] = x_ref[...] + jnp.where(hit, jnp.float32(1.0), jnp.float32(0.0))


def kernel(x):
    n, s, d = x.shape
    return pl.pallas_call(
        _scatter_add_body,
        out_shape=jax.ShapeDtypeStruct((n, s, d), x.dtype),
        grid=(1,),
        in_specs=[pl.BlockSpec((2, s, d), lambda i: (0, 0, 0))],
        out_specs=pl.BlockSpec((2, s, d), lambda i: (0, 0, 0)),
        input_output_aliases={0: 0},
    )(x)


# speedup vs baseline: 7.2701x; 3.1723x over previous
import jax
import jax.numpy as jnp
from jax import lax
from jax.experimental import pallas as pl

_BLK = 512


def _copy_scatter_body(x_ref, o_ref):
    o_ref[...] = x_ref[...]

    @pl.when(pl.program_id(0) == 0)
    def _():
        blk = o_ref[0:2, 0:8, :]
        i0 = lax.broadcasted_iota(jnp.int32, (2, 8, 1024), 0)
        i1 = lax.broadcasted_iota(jnp.int32, (2, 8, 1024), 1)
        hit = ((i0 == 0) & (i1 == 0)) | ((i0 == 1) & (i1 == 1))
        o_ref[0:2, 0:8, :] = blk + jnp.where(hit, jnp.float32(1.0), jnp.float32(0.0))


def kernel(x):
    n, s, d = x.shape
    xt = jnp.transpose(x, (1, 0, 2))
    out_t = pl.pallas_call(
        _copy_scatter_body,
        out_shape=jax.ShapeDtypeStruct((s, n, d), x.dtype),
        grid=(n // _BLK,),
        in_specs=[pl.BlockSpec((s, _BLK, d), lambda i: (0, i, 0))],
        out_specs=pl.BlockSpec((s, _BLK, d), lambda i: (0, i, 0)),
    )(xt)
    return jnp.transpose(out_t, (1, 0, 2))


# speedup vs baseline: 7.3128x; 1.0059x over previous
import jax
import jax.numpy as jnp
from jax import lax
from jax.experimental import pallas as pl

_BLK = 1024


def _copy_scatter_body(x_ref, o_ref):
    o_ref[...] = x_ref[...]

    @pl.when(pl.program_id(0) == 0)
    def _():
        blk = o_ref[0:2, 0:8, :]
        i0 = lax.broadcasted_iota(jnp.int32, (2, 8, 1024), 0)
        i1 = lax.broadcasted_iota(jnp.int32, (2, 8, 1024), 1)
        hit = ((i0 == 0) & (i1 == 0)) | ((i0 == 1) & (i1 == 1))
        o_ref[0:2, 0:8, :] = blk + jnp.where(hit, jnp.float32(1.0), jnp.float32(0.0))


def kernel(x):
    n, s, d = x.shape
    xt = jnp.transpose(x, (1, 0, 2))
    out_t = pl.pallas_call(
        _copy_scatter_body,
        out_shape=jax.ShapeDtypeStruct((s, n, d), x.dtype),
        grid=(n // _BLK,),
        in_specs=[pl.BlockSpec((s, _BLK, d), lambda i: (0, i, 0))],
        out_specs=pl.BlockSpec((s, _BLK, d), lambda i: (0, i, 0)),
    )(xt)
    return jnp.transpose(out_t, (1, 0, 2))
